# Initial kernel scaffold; baseline (speedup 1.0000x reference)
import jax
import jax.numpy as jnp
from jax.experimental import pallas as pl


def kernel(img_batch, W1, b1, W2, b2, W3, b3, W4, b4, W5, b5, Px1, Pa1, Px2, Pa2, Px3, Pa3, Px4, Pa4, Px5, Pa5, R1, rb1, R2, rb2, R3, rb3):
    raise NotImplementedError("write your pallas kernel here")


# speedup vs baseline: 25.2497x; 25.2497x over previous
import jax
import jax.numpy as jnp
from jax.experimental import pallas as pl

KSZ = 5
PAD = KSZ // 2


def _shift(padded, dy, dx, S):
    if padded.ndim == 3:
        return padded[:, PAD + dy:PAD + dy + S, PAD + dx:PAD + dx + S]
    return padded[:, PAD + dy:PAD + dy + S, PAD + dx:PAD + dx + S, :]


def _avgpool4(x):
    B, S, _, C = x.shape
    r = x.reshape(B, S // 2, 2, S // 2, 2, C)
    return r.mean(axis=(2, 4))


def _avgpool3(x):
    B, S, _ = x.shape
    r = x.reshape(B, S // 2, 2, S // 2, 2)
    return r.mean(axis=(2, 4))


def _gated_conv(x4, pmy, pmx, Wf, b2):
    B, S, _, ci = x4.shape
    pads = ((0, 0), (PAD, PAD), (PAD, PAD))
    xp = jnp.pad(x4, pads + ((0, 0),))
    pyp = jnp.pad(pmy, pads)
    pxp = jnp.pad(pmx, pads)
    pieces = []
    for dy in range(-PAD, PAD + 1):
        for dx in range(-PAD, PAD + 1):
            xs = _shift(xp, dy, dx, S)
            dpy = pmy - _shift(pyp, dy, dx, S)
            dpx = pmx - _shift(pxp, dy, dx, S)
            gate = jnp.exp(-(dpy * dpy + dpx * dpx))
            pieces.append(xs * gate[..., None])
    U = jnp.concatenate(pieces, axis=-1)
    h = jnp.dot(U.reshape(B * S * S, 25 * ci), Wf,
                preferred_element_type=jnp.float32)
    return jax.nn.relu(h + b2)


def _layer(x4, a4, pmy, pmx, Wf, b2, Px, Pa, mult_a):
    B, S, _, _ = x4.shape
    S2 = S // 2
    co = Wf.shape[1]
    h = _gated_conv(x4, pmy, pmx, Wf, b2)
    if mult_a:
        h = h * a4.reshape(B * S * S, -1)
    h4 = h.reshape(B, S, S, co)
    xg = _avgpool4(h4)
    ag = _avgpool4(jax.nn.relu(a4))
    pmy2 = _avgpool3(pmy)
    pmx2 = _avgpool3(pmx)
    x_new = jax.nn.relu(
        jnp.dot(xg.reshape(B * S2 * S2, co), Px,
                preferred_element_type=jnp.float32))
    a_new = jax.nn.sigmoid(
        jnp.dot(ag.reshape(B * S2 * S2, -1), Pa,
                preferred_element_type=jnp.float32))
    fo = Px.shape[1]
    ao = Pa.shape[1]
    return (x_new.reshape(B, S2, S2, fo), a_new.reshape(B, S2, S2, ao),
            pmy2, pmx2, xg)


def _conv_body(img_ref,
               Wf1_ref, b1_ref, Wf2_ref, b2_ref, Wf3_ref, b3_ref,
               Wf4_ref, b4_ref, Wf5_ref, b5_ref,
               Px1_ref, Pa1_ref, Px2_ref, Pa2_ref, Px3_ref, Pa3_ref,
               Px4_ref, Pa4_ref, Px5_ref, Pa5_ref,
               lsm_ref, aout_ref, pg_ref, xg_ref):
    img = img_ref[:]
    B, S, _ = img.shape

    gx = jnp.pad(img[:, :, 1:] - img[:, :, :-1], ((0, 0), (0, 0), (0, 1)))
    gy = jnp.pad(img[:, 1:, :] - img[:, :-1, :], ((0, 0), (0, 1), (0, 0)))
    a4 = jnp.sqrt(gx * gx + gy * gy + 1e-12)[..., None]
    pmy, pmx = gy, gx

    x4 = img[..., None]
    convs = [(Wf1_ref[:], b1_ref[:]), (Wf2_ref[:], b2_ref[:]),
             (Wf3_ref[:], b3_ref[:]), (Wf4_ref[:], b4_ref[:]),
             (Wf5_ref[:], b5_ref[:])]
    pools = [(Px1_ref[:], Pa1_ref[:]), (Px2_ref[:], Pa2_ref[:]),
             (Px3_ref[:], Pa3_ref[:]), (Px4_ref[:], Pa4_ref[:]),
             (Px5_ref[:], Pa5_ref[:])]
    xg = None
    for i in range(5):
        Wf, b2 = convs[i]
        Px, Pa = pools[i]
        x4, a4, pmy, pmx, xg = _layer(x4, a4, pmy, pmx, Wf, b2, Px, Pa,
                                      mult_a=(i > 0))

    logits = x4.reshape(B, 10)
    m = jnp.max(logits, axis=1, keepdims=True)
    z = logits - m
    lsm_ref[:] = (z - jnp.log(jnp.sum(jnp.exp(z), axis=1,
                                      keepdims=True))).reshape(B, 1, 10)
    aout_ref[:] = a4.reshape(B, 1, 10)
    pg_ref[:] = jnp.concatenate([pmy.reshape(B, 1), pmx.reshape(B, 1)],
                                axis=1).reshape(B, 1, 2)
    xg_ref[:] = xg.reshape(B, 1, 64)


def _mlp_body(xg_ref, a_ref, pg_ref, R1_ref, rb1_ref, R2_ref, rb2_ref,
              R3_ref, rb3_ref, rec_ref):
    B = xg_ref.shape[0]
    pg = pg_ref[:]
    pose_flat = jnp.concatenate(
        [pg[:, 0:1], pg[:, 1:2]] * 10, axis=1)
    rec_in = jnp.concatenate([xg_ref[:], a_ref[:], pose_flat], axis=1)
    r1 = jax.nn.relu(jnp.dot(rec_in, R1_ref[:],
                             preferred_element_type=jnp.float32) + rb1_ref[:])
    r2 = jax.nn.relu(jnp.dot(r1, R2_ref[:],
                             preferred_element_type=jnp.float32) + rb2_ref[:])
    rec_ref[:] = jax.nn.sigmoid(
        jnp.dot(r2, R3_ref[:], preferred_element_type=jnp.float32)
        + rb3_ref[:])


def kernel(img_batch, W1, b1, W2, b2, W3, b3, W4, b4, W5, b5,
           Px1, Pa1, Px2, Pa2, Px3, Pa3, Px4, Pa4, Px5, Pa5,
           R1, rb1, R2, rb2, R3, rb3):
    B, S = img_batch.shape[0], img_batch.shape[1]
    f32 = jnp.float32
    img3 = img_batch[..., 0]

    flat_w = []
    flat_specs = []
    for W, b in ((W1, b1), (W2, b2), (W3, b3), (W4, b4), (W5, b5)):
        k, ci, co = W.shape
        flat_w.append(W.reshape(k * ci, co))
        flat_w.append(b.reshape(1, co))
        flat_specs.append(pl.BlockSpec((k * ci, co), lambda b_: (0, 0)))
        flat_specs.append(pl.BlockSpec((1, co), lambda b_: (0, 0)))
    pool_w = [Px1, Pa1, Px2, Pa2, Px3, Pa3, Px4, Pa4, Px5, Pa5]
    pool_specs = [pl.BlockSpec(p.shape, lambda b_: (0, 0)) for p in pool_w]

    lsm, a_out, pg, xg5 = pl.pallas_call(
        _conv_body,
        grid=(B,),
        in_specs=[pl.BlockSpec((1, S, S), lambda b_: (b_, 0, 0))]
                 + flat_specs + pool_specs,
        out_specs=(pl.BlockSpec((1, 1, 10), lambda b_: (b_, 0, 0)),
                   pl.BlockSpec((1, 1, 10), lambda b_: (b_, 0, 0)),
                   pl.BlockSpec((1, 1, 2), lambda b_: (b_, 0, 0)),
                   pl.BlockSpec((1, 1, 64), lambda b_: (b_, 0, 0))),
        out_shape=(
            jax.ShapeDtypeStruct((B, 1, 10), f32),
            jax.ShapeDtypeStruct((B, 1, 10), f32),
            jax.ShapeDtypeStruct((B, 1, 2), f32),
            jax.ShapeDtypeStruct((B, 1, 64), f32),
        ),
    )(img3, *flat_w, *pool_w)
    lsm = lsm.reshape(B, 10)
    a_out = a_out.reshape(B, 10)
    pg = pg.reshape(B, 2)
    xg5 = xg5.reshape(B, 64)

    rec = pl.pallas_call(
        _mlp_body,
        out_shape=jax.ShapeDtypeStruct((B, 784), f32),
    )(xg5, a_out, pg, R1, rb1.reshape(1, -1), R2, rb2.reshape(1, -1),
      R3, rb3.reshape(1, -1))

    pose = jnp.broadcast_to(pg.reshape(B, 1, 1, 1, 2), (B, 1, 1, 10, 2))
    return (lsm, a_out, pose, rec)


# speedup vs baseline: 206.2357x; 8.1679x over previous
import jax
import jax.numpy as jnp
from jax.experimental import pallas as pl

BX = 1024
NIMG = 32
SIMG = 32
F32 = jnp.float32

CONV_DIMS = [(1, 16), (16, 16), (32, 32), (32, 32), (64, 64)]


def _pshift(P, dy, dl):
    Y, L = P.shape
    P2 = jnp.pad(P, ((max(-dy, 0), max(dy, 0)), (max(-dl, 0), max(dl, 0))))
    return P2[max(dy, 0):max(dy, 0) + Y, max(dl, 0):max(dl, 0) + L]


def _lshift(t, d):
    C, N = t.shape
    t2 = jnp.pad(t, ((0, 0), (max(-d, 0), max(d, 0))))
    return t2[:, max(d, 0):max(d, 0) + N]


def _pool_ch(t, Y, s):
    tx = t + _lshift(t, s)
    ty = tx + _lshift(tx, BX)
    rows = [ty[:, (2 * j) * BX:(2 * j) * BX + BX] for j in range(Y // 2)]
    return jnp.concatenate(rows, axis=1) * 0.25


def _pool_plane(P, s):
    Y = P.shape[0]
    Px_ = P + _pshift(P, 0, s)
    Py_ = Px_ + _pshift(Px_, 1, 0)
    rows = [Py_[2 * j:2 * j + 1] for j in range(Y // 2)]
    return jnp.concatenate(rows, axis=0) * 0.25


def _body(imgP_ref,
          Wt1_ref, b1_ref, Wt2_ref, b2_ref, Wt3_ref, b3_ref,
          Wt4_ref, b4_ref, Wt5_ref, b5_ref,
          Px1_ref, Pa1_ref, Px2_ref, Pa2_ref, Px3_ref, Pa3_ref,
          Px4_ref, Pa4_ref, Px5_ref, Pa5_ref,
          R1_ref, rb1_ref, R2_ref, rb2_ref, R3_ref, rb3_ref, E_ref,
          lsm_ref, aout_ref, pg_ref, rec_ref):
    imgP = imgP_ref[:]
    xposP = jax.lax.broadcasted_iota(jnp.int32, (1, BX), 1) % SIMG
    yposP = jax.lax.broadcasted_iota(jnp.int32, (SIMG, 1), 0)

    gxP = jnp.where(xposP < SIMG - 1, _pshift(imgP, 0, 1) - imgP, 0.0)
    gyP = jnp.where(yposP < SIMG - 1, _pshift(imgP, 1, 0) - imgP, 0.0)
    aP = jnp.sqrt(gxP * gxP + gyP * gyP + 1e-12)
    pmy, pmx = gyP, gxP

    wts = [Wt1_ref[:], Wt2_ref[:], Wt3_ref[:], Wt4_ref[:], Wt5_ref[:]]
    bs = [b1_ref[:], b2_ref[:], b3_ref[:], b4_ref[:], b5_ref[:]]
    pxs = [Px1_ref[:], Px2_ref[:], Px3_ref[:], Px4_ref[:], Px5_ref[:]]
    pas = [Pa1_ref[:], Pa2_ref[:], Pa3_ref[:], Pa4_ref[:], Pa5_ref[:]]

    X = None
    A = None
    xg = None
    for i in range(5):
        s = 1 << i
        Y = SIMG >> i
        N = Y * BX
        ci, co = CONV_DIMS[i]
        Wt = wts[i]
        xpos = jax.lax.broadcasted_iota(jnp.int32, (1, N), 1) % SIMG
        h = jnp.broadcast_to(bs[i], (co, N))
        for dy in range(-2, 3):
            if dy <= -Y or dy >= Y:
                continue
            for dx in range(-2, 3):
                if dx * s <= -SIMG or dx * s >= SIMG:
                    continue
                k = (dy + 2) * 5 + (dx + 2)
                dpy = pmy - _pshift(pmy, dy, dx * s)
                dpx = pmx - _pshift(pmx, dy, dx * s)
                gP = jnp.exp(-(dpy * dpy + dpx * dpx))
                xv = xpos + dx * s
                mask = (xv >= 0) & (xv < SIMG)
                if i == 0:
                    pieceP = gP * _pshift(imgP, dy, dx * s)
                    piece = jnp.where(mask, pieceP.reshape(1, N), 0.0)
                else:
                    g = jnp.where(mask, gP.reshape(1, N), 0.0)
                    piece = _lshift(X, dy * BX + dx * s) * g
                h = h + jnp.dot(Wt[k * co:(k + 1) * co, :], piece,
                                preferred_element_type=F32)
        h = jax.nn.relu(h)
        if i > 0:
            h = h * A
        xg = _pool_ch(h, Y, s)
        if i == 0:
            agP = _pool_plane(jax.nn.relu(aP), s)
            ag = agP.reshape(1, N // 2)
        else:
            ag = _pool_ch(jax.nn.relu(A), Y, s)
        pmy = _pool_plane(pmy, s)
        pmx = _pool_plane(pmx, s)
        X = jax.nn.relu(jnp.dot(pxs[i], xg, preferred_element_type=F32))
        A = jax.nn.sigmoid(jnp.dot(pas[i], ag, preferred_element_type=F32))

    E = E_ref[:]
    logits = jnp.dot(X, E, preferred_element_type=F32)
    m = jnp.max(logits, axis=0, keepdims=True)
    z = logits - m
    lsm = z - jnp.log(jnp.sum(jnp.exp(z), axis=0, keepdims=True))
    aT = jnp.dot(A, E, preferred_element_type=F32)
    xgT = jnp.dot(xg, E, preferred_element_type=F32)
    pgy = jnp.dot(pmy.reshape(1, BX), E, preferred_element_type=F32)
    pgx = jnp.dot(pmx.reshape(1, BX), E, preferred_element_type=F32)

    rec_in = jnp.concatenate([xgT, aT] + [pgy, pgx] * 10, axis=0)
    r1 = jax.nn.relu(jnp.dot(R1_ref[:], rec_in,
                             preferred_element_type=F32) + rb1_ref[:])
    r2 = jax.nn.relu(jnp.dot(R2_ref[:], r1,
                             preferred_element_type=F32) + rb2_ref[:])
    r3 = jax.nn.sigmoid(jnp.dot(R3_ref[:], r2,
                                preferred_element_type=F32) + rb3_ref[:])

    lsm_ref[:] = lsm
    aout_ref[:] = aT
    pg_ref[:] = jnp.concatenate([pgy, pgx], axis=0)
    rec_ref[:] = r3


def kernel(img_batch, W1, b1, W2, b2, W3, b3, W4, b4, W5, b5,
           Px1, Pa1, Px2, Pa2, Px3, Pa3, Px4, Pa4, Px5, Pa5,
           R1, rb1, R2, rb2, R3, rb3):
    B = img_batch.shape[0]
    imgP = img_batch[..., 0].transpose(1, 0, 2).reshape(SIMG, BX)

    flat_w = []
    for W, b in ((W1, b1), (W2, b2), (W3, b3), (W4, b4), (W5, b5)):
        k, ci, co = W.shape
        flat_w.append(W.transpose(0, 2, 1).reshape(k * co, ci))
        flat_w.append(b.reshape(co, 1))
    pool_w = [Px1.T, Pa1.T, Px2.T, Pa2.T, Px3.T, Pa3.T, Px4.T, Pa4.T,
              Px5.T, Pa5.T]
    n = jnp.arange(BX)[:, None]
    E = (n == (jnp.arange(NIMG) * SIMG)[None, :]).astype(F32)

    lsm_t, a_t, pg_t, rec_t = pl.pallas_call(
        _body,
        out_shape=(
            jax.ShapeDtypeStruct((10, NIMG), F32),
            jax.ShapeDtypeStruct((10, NIMG), F32),
            jax.ShapeDtypeStruct((2, NIMG), F32),
            jax.ShapeDtypeStruct((784, NIMG), F32),
        ),
    )(imgP, *flat_w, *pool_w,
      R1.T, rb1.reshape(-1, 1), R2.T, rb2.reshape(-1, 1),
      R3.T, rb3.reshape(-1, 1), E)

    pg = pg_t.T
    pose = jnp.broadcast_to(pg.reshape(B, 1, 1, 1, 2), (B, 1, 1, 10, 2))
    return (lsm_t.T, a_t.T, pose, rec_t.T)


# speedup vs baseline: 218.0736x; 1.0574x over previous
import jax
import jax.numpy as jnp
from jax.experimental import pallas as pl

BX = 1024
NIMG = 32
SIMG = 32
F32 = jnp.float32

CONV_DIMS = [(1, 16), (16, 16), (32, 32), (32, 32), (64, 64)]


def _pshift(P, dy, dl):
    Y, L = P.shape
    P2 = jnp.pad(P, ((max(-dy, 0), max(dy, 0)), (max(-dl, 0), max(dl, 0))))
    return P2[max(dy, 0):max(dy, 0) + Y, max(dl, 0):max(dl, 0) + L]


def _lshift(t, d):
    C, N = t.shape
    t2 = jnp.pad(t, ((0, 0), (max(-d, 0), max(d, 0))))
    return t2[:, max(d, 0):max(d, 0) + N]


def _pool_ch(t, Y, s):
    tx = t + _lshift(t, s)
    ty = tx + _lshift(tx, BX)
    rows = [ty[:, (2 * j) * BX:(2 * j) * BX + BX] for j in range(Y // 2)]
    return jnp.concatenate(rows, axis=1) * 0.25


def _pool_plane(P, s):
    Y = P.shape[0]
    Px_ = P + _pshift(P, 0, s)
    Py_ = Px_ + _pshift(Px_, 1, 0)
    rows = [Py_[2 * j:2 * j + 1] for j in range(Y // 2)]
    return jnp.concatenate(rows, axis=0) * 0.25


def _body(imgP_ref,
          Wt1_ref, b1_ref, Wt2_ref, b2_ref, Wt3_ref, b3_ref,
          Wt4_ref, b4_ref, Wt5_ref, b5_ref,
          Px1_ref, Pa1_ref, Px2_ref, Pa2_ref, Px3_ref, Pa3_ref,
          Px4_ref, Pa4_ref, Px5_ref, Pa5_ref,
          R1_ref, rb1_ref, R2_ref, rb2_ref, R3_ref, rb3_ref, E_ref,
          lsm_ref, aout_ref, pg_ref, rec_ref):
    imgP = imgP_ref[:]
    xposP = jax.lax.broadcasted_iota(jnp.int32, (1, BX), 1) % SIMG
    yposP = jax.lax.broadcasted_iota(jnp.int32, (SIMG, 1), 0)

    gxP = jnp.where(xposP < SIMG - 1, _pshift(imgP, 0, 1) - imgP, 0.0)
    gyP = jnp.where(yposP < SIMG - 1, _pshift(imgP, 1, 0) - imgP, 0.0)
    aP = jnp.sqrt(gxP * gxP + gyP * gyP + 1e-12)
    pmy, pmx = gyP, gxP

    wts = [Wt1_ref[:], Wt2_ref[:], Wt3_ref[:], Wt4_ref[:], Wt5_ref[:]]
    bs = [b1_ref[:], b2_ref[:], b3_ref[:], b4_ref[:], b5_ref[:]]
    pxs = [Px1_ref[:], Px2_ref[:], Px3_ref[:], Px4_ref[:], Px5_ref[:]]
    pas = [Pa1_ref[:], Pa2_ref[:], Pa3_ref[:], Pa4_ref[:], Pa5_ref[:]]

    X = None
    A = None
    xg = None
    for i in range(5):
        s = 1 << i
        Y = SIMG >> i
        N = Y * BX
        ci, co = CONV_DIMS[i]
        Wt = wts[i]
        h = jnp.broadcast_to(bs[i], (co, N))
        for dy in range(-2, 3):
            if dy <= -Y or dy >= Y:
                continue
            for dx in range(-2, 3):
                if dx * s <= -SIMG or dx * s >= SIMG:
                    continue
                k = (dy + 2) * 5 + (dx + 2)
                Wk = Wt[k * co:(k + 1) * co, :]
                if dy == 0 and dx == 0:
                    piece = imgP.reshape(1, N) if i == 0 else X
                    h = h + jnp.dot(Wk, piece, preferred_element_type=F32)
                    continue
                dpy = pmy - _pshift(pmy, dy, dx * s)
                dpx = pmx - _pshift(pmx, dy, dx * s)
                gP = jnp.exp(-(dpy * dpy + dpx * dpx))
                if dx != 0:
                    xv = xposP + dx * s
                    gP = jnp.where((xv >= 0) & (xv < SIMG), gP, 0.0)
                if i == 0:
                    piece = (gP * _pshift(imgP, dy, dx * s)).reshape(1, N)
                    h = h + jnp.dot(Wk, piece, preferred_element_type=F32)
                else:
                    mk = jnp.dot(Wk, _lshift(X, dy * BX + dx * s),
                                 preferred_element_type=F32)
                    h = h + gP.reshape(1, N) * mk
        h = jax.nn.relu(h)
        if i > 0:
            h = h * A
        xg = _pool_ch(h, Y, s)
        if i == 0:
            agP = _pool_plane(jax.nn.relu(aP), s)
            ag = agP.reshape(1, N // 2)
        else:
            ag = _pool_ch(jax.nn.relu(A), Y, s)
        pmy = _pool_plane(pmy, s)
        pmx = _pool_plane(pmx, s)
        X = jax.nn.relu(jnp.dot(pxs[i], xg, preferred_element_type=F32))
        A = jax.nn.sigmoid(jnp.dot(pas[i], ag, preferred_element_type=F32))

    E = E_ref[:]
    logits = jnp.dot(X, E, preferred_element_type=F32)
    m = jnp.max(logits, axis=0, keepdims=True)
    z = logits - m
    lsm = z - jnp.log(jnp.sum(jnp.exp(z), axis=0, keepdims=True))
    aT = jnp.dot(A, E, preferred_element_type=F32)
    xgT = jnp.dot(xg, E, preferred_element_type=F32)
    pgy = jnp.dot(pmy.reshape(1, BX), E, preferred_element_type=F32)
    pgx = jnp.dot(pmx.reshape(1, BX), E, preferred_element_type=F32)

    rec_in = jnp.concatenate([xgT, aT] + [pgy, pgx] * 10, axis=0)
    r1 = jax.nn.relu(jnp.dot(R1_ref[:], rec_in,
                             preferred_element_type=F32) + rb1_ref[:])
    r2 = jax.nn.relu(jnp.dot(R2_ref[:], r1,
                             preferred_element_type=F32) + rb2_ref[:])
    r3 = jax.nn.sigmoid(jnp.dot(R3_ref[:], r2,
                                preferred_element_type=F32) + rb3_ref[:])

    lsm_ref[:] = lsm
    aout_ref[:] = aT
    pg_ref[:] = jnp.concatenate([pgy, pgx], axis=0)
    rec_ref[:] = r3


def kernel(img_batch, W1, b1, W2, b2, W3, b3, W4, b4, W5, b5,
           Px1, Pa1, Px2, Pa2, Px3, Pa3, Px4, Pa4, Px5, Pa5,
           R1, rb1, R2, rb2, R3, rb3):
    B = img_batch.shape[0]
    imgP = img_batch[..., 0].transpose(1, 0, 2).reshape(SIMG, BX)

    flat_w = []
    for W, b in ((W1, b1), (W2, b2), (W3, b3), (W4, b4), (W5, b5)):
        k, ci, co = W.shape
        flat_w.append(W.transpose(0, 2, 1).reshape(k * co, ci))
        flat_w.append(b.reshape(co, 1))
    pool_w = [Px1.T, Pa1.T, Px2.T, Pa2.T, Px3.T, Pa3.T, Px4.T, Pa4.T,
              Px5.T, Pa5.T]
    n = jnp.arange(BX)[:, None]
    E = (n == (jnp.arange(NIMG) * SIMG)[None, :]).astype(F32)

    lsm_t, a_t, pg_t, rec_t = pl.pallas_call(
        _body,
        out_shape=(
            jax.ShapeDtypeStruct((10, NIMG), F32),
            jax.ShapeDtypeStruct((10, NIMG), F32),
            jax.ShapeDtypeStruct((2, NIMG), F32),
            jax.ShapeDtypeStruct((784, NIMG), F32),
        ),
    )(imgP, *flat_w, *pool_w,
      R1.T, rb1.reshape(-1, 1), R2.T, rb2.reshape(-1, 1),
      R3.T, rb3.reshape(-1, 1), E)

    pg = pg_t.T
    pose = jnp.broadcast_to(pg.reshape(B, 1, 1, 1, 2), (B, 1, 1, 10, 2))
    return (lsm_t.T, a_t.T, pose, rec_t.T)


# speedup vs baseline: 232.7236x; 1.0672x over previous
import jax
import jax.numpy as jnp
from jax.experimental import pallas as pl

BX = 1024
NIMG = 32
SIMG = 32
F32 = jnp.float32

CONV_DIMS = [(1, 16), (16, 16), (32, 32), (32, 32), (64, 64)]


def _pshift(P, dy, dl):
    Y, L = P.shape
    P2 = jnp.pad(P, ((max(-dy, 0), max(dy, 0)), (max(-dl, 0), max(dl, 0))))
    return P2[max(dy, 0):max(dy, 0) + Y, max(dl, 0):max(dl, 0) + L]


def _lshift(t, d):
    C, N = t.shape
    t2 = jnp.pad(t, ((0, 0), (max(-d, 0), max(d, 0))))
    return t2[:, max(d, 0):max(d, 0) + N]


def _pool_ch(t, Y, s):
    tx = t + _lshift(t, s)
    ty = tx + _lshift(tx, BX)
    rows = [ty[:, (2 * j) * BX:(2 * j) * BX + BX] for j in range(Y // 2)]
    return jnp.concatenate(rows, axis=1)


def _pool_plane(P, s, scale=True):
    Y = P.shape[0]
    Px_ = P + _pshift(P, 0, s)
    Py_ = Px_ + _pshift(Px_, 1, 0)
    rows = [Py_[2 * j:2 * j + 1] for j in range(Y // 2)]
    out = jnp.concatenate(rows, axis=0)
    return out * 0.25 if scale else out


def _body(imgP_ref,
          Wt1_ref, b1_ref, Wt2_ref, b2_ref, Wt3_ref, b3_ref,
          Wt4_ref, b4_ref, Wt5_ref, b5_ref,
          Px1_ref, Pa1_ref, Px2_ref, Pa2_ref, Px3_ref, Pa3_ref,
          Px4_ref, Pa4_ref, Px5_ref, Pa5_ref,
          R1_ref, rb1_ref, R2_ref, rb2_ref, R3_ref, rb3_ref, E_ref,
          lsm_ref, aout_ref, pg_ref, rec_ref):
    imgP = imgP_ref[:]
    xposP = jax.lax.broadcasted_iota(jnp.int32, (1, BX), 1) % SIMG
    yposP = jax.lax.broadcasted_iota(jnp.int32, (SIMG, 1), 0)

    gxP = jnp.where(xposP < SIMG - 1, _pshift(imgP, 0, 1) - imgP, 0.0)
    gyP = jnp.where(yposP < SIMG - 1, _pshift(imgP, 1, 0) - imgP, 0.0)
    aP = jnp.sqrt(gxP * gxP + gyP * gyP + 1e-12)
    pmy, pmx = gyP, gxP

    wts = [Wt1_ref[:], Wt2_ref[:], Wt3_ref[:], Wt4_ref[:], Wt5_ref[:]]
    bs = [b1_ref[:], b2_ref[:], b3_ref[:], b4_ref[:], b5_ref[:]]
    pxs = [Px1_ref[:], Px2_ref[:], Px3_ref[:], Px4_ref[:], Px5_ref[:]]
    pas = [Pa1_ref[:], Pa2_ref[:], Pa3_ref[:], Pa4_ref[:], Pa5_ref[:]]

    X = None
    A = None
    xg = None
    for i in range(5):
        s = 1 << i
        Y = SIMG >> i
        N = Y * BX
        ci, co = CONV_DIMS[i]
        Wt = wts[i]
        h = jnp.broadcast_to(bs[i], (co, N))
        for dy in range(-2, 3):
            if dy <= -Y or dy >= Y:
                continue
            for dx in range(-2, 3):
                if dx * s <= -SIMG or dx * s >= SIMG:
                    continue
                k = (dy + 2) * 5 + (dx + 2)
                Wk = Wt[k * co:(k + 1) * co, :]
                if dy == 0 and dx == 0:
                    piece = imgP.reshape(1, N) if i == 0 else X
                    h = h + jnp.dot(Wk, piece, preferred_element_type=F32)
                    continue
                dpy = pmy - _pshift(pmy, dy, dx * s)
                dpx = pmx - _pshift(pmx, dy, dx * s)
                gP = jnp.exp(-(dpy * dpy + dpx * dpx))
                if dx != 0:
                    xv = xposP + dx * s
                    gP = jnp.where((xv >= 0) & (xv < SIMG), gP, 0.0)
                if i == 0:
                    piece = (gP * _pshift(imgP, dy, dx * s)).reshape(1, N)
                    h = h + jnp.dot(Wk, piece, preferred_element_type=F32)
                else:
                    mk = jnp.dot(Wk, _lshift(X, dy * BX + dx * s),
                                 preferred_element_type=F32)
                    h = h + gP.reshape(1, N) * mk
        h = jax.nn.relu(h)
        if i > 0:
            h = h * A
        xg = _pool_ch(h, Y, s)
        if i == 4:
            xg = xg * 0.25
        if i == 0:
            agP = _pool_plane(jax.nn.relu(aP), s, scale=False)
            ag = agP.reshape(1, N // 2)
        else:
            ag = _pool_ch(jax.nn.relu(A), Y, s)
        pmy = _pool_plane(pmy, s)
        pmx = _pool_plane(pmx, s)
        X = jax.nn.relu(jnp.dot(pxs[i], xg, preferred_element_type=F32))
        A = jax.nn.sigmoid(jnp.dot(pas[i], ag, preferred_element_type=F32))

    E = E_ref[:]
    logits = jnp.dot(X, E, preferred_element_type=F32)
    m = jnp.max(logits, axis=0, keepdims=True)
    z = logits - m
    lsm = z - jnp.log(jnp.sum(jnp.exp(z), axis=0, keepdims=True))
    aT = jnp.dot(A, E, preferred_element_type=F32)
    xgT = jnp.dot(xg, E, preferred_element_type=F32)
    pgy = jnp.dot(pmy.reshape(1, BX), E, preferred_element_type=F32)
    pgx = jnp.dot(pmx.reshape(1, BX), E, preferred_element_type=F32)

    rec_in = jnp.concatenate([xgT, aT] + [pgy, pgx] * 10, axis=0)
    rec_inT = rec_in.T
    r1 = jax.nn.relu(jnp.dot(rec_inT, R1_ref[:],
                             preferred_element_type=F32) + rb1_ref[:])
    r2 = jax.nn.relu(jnp.dot(r1, R2_ref[:],
                             preferred_element_type=F32) + rb2_ref[:])
    r3 = jax.nn.sigmoid(jnp.dot(r2, R3_ref[:],
                                preferred_element_type=F32) + rb3_ref[:])

    lsm_ref[:] = lsm
    aout_ref[:] = aT
    pg_ref[:] = jnp.concatenate([pgy, pgx], axis=0)
    rec_ref[:] = r3


def kernel(img_batch, W1, b1, W2, b2, W3, b3, W4, b4, W5, b5,
           Px1, Pa1, Px2, Pa2, Px3, Pa3, Px4, Pa4, Px5, Pa5,
           R1, rb1, R2, rb2, R3, rb3):
    B = img_batch.shape[0]
    imgP = img_batch[..., 0].transpose(1, 0, 2).reshape(SIMG, BX)

    flat_w = []
    for W, b in ((W1, b1), (W2, b2), (W3, b3), (W4, b4), (W5, b5)):
        k, ci, co = W.shape
        flat_w.append(W.transpose(0, 2, 1).reshape(k * co, ci))
        flat_w.append(b.reshape(co, 1))
    pool_w = [Px1.T * 0.25, Pa1.T * 0.25, Px2.T * 0.25, Pa2.T * 0.25,
              Px3.T * 0.25, Pa3.T * 0.25, Px4.T * 0.25, Pa4.T * 0.25,
              Px5.T, Pa5.T * 0.25]
    n = jnp.arange(BX)[:, None]
    E = (n == (jnp.arange(NIMG) * SIMG)[None, :]).astype(F32)

    lsm_t, a_t, pg_t, rec = pl.pallas_call(
        _body,
        out_shape=(
            jax.ShapeDtypeStruct((10, NIMG), F32),
            jax.ShapeDtypeStruct((10, NIMG), F32),
            jax.ShapeDtypeStruct((2, NIMG), F32),
            jax.ShapeDtypeStruct((NIMG, 784), F32),
        ),
    )(imgP, *flat_w, *pool_w,
      R1, rb1.reshape(1, -1), R2, rb2.reshape(1, -1),
      R3, rb3.reshape(1, -1), E)

    pg = pg_t.T
    pose = jnp.broadcast_to(pg.reshape(B, 1, 1, 1, 2), (B, 1, 1, 10, 2))
    return (lsm_t.T, a_t.T, pose, rec)


# speedup vs baseline: 237.9347x; 1.0224x over previous
import jax
import jax.numpy as jnp
from jax.experimental import pallas as pl
from jax.experimental.pallas import tpu as pltpu

BX = 1024
NIMG = 32
SIMG = 32
F32 = jnp.float32

CONV_DIMS = [(1, 16), (16, 16), (32, 32), (32, 32), (64, 64)]


def _pshift(P, dy, dl):
    Y, L = P.shape
    P2 = jnp.pad(P, ((max(-dy, 0), max(dy, 0)), (max(-dl, 0), max(dl, 0))))
    return P2[max(dy, 0):max(dy, 0) + Y, max(dl, 0):max(dl, 0) + L]


def _lshift(t, d):
    C, N = t.shape
    t2 = jnp.pad(t, ((0, 0), (max(-d, 0), max(d, 0))))
    return t2[:, max(d, 0):max(d, 0) + N]


def _pool_ch(t, Y, s):
    tx = t + _lshift(t, s)
    ty = tx + _lshift(tx, BX)
    rows = [ty[:, (2 * j) * BX:(2 * j) * BX + BX] for j in range(Y // 2)]
    return jnp.concatenate(rows, axis=1)


def _pool_plane(P, s, scale=True):
    Y = P.shape[0]
    Px_ = P + _pshift(P, 0, s)
    Py_ = Px_ + _pshift(Px_, 1, 0)
    rows = [Py_[2 * j:2 * j + 1] for j in range(Y // 2)]
    out = jnp.concatenate(rows, axis=0)
    return out * 0.25 if scale else out


def _body(imgP_ref,
          Wt1_ref, b1_ref, Wt2_ref, b2_ref, Wt3_ref, b3_ref,
          Wt4_ref, b4_ref, Wt5_ref, b5_ref,
          Px1_ref, Pa1_ref, Px2_ref, Pa2_ref, Px3_ref, Pa3_ref,
          Px4_ref, Pa4_ref, Px5_ref, Pa5_ref,
          R1_ref, rb1_ref, R2_ref, rb2_ref, R3_ref, rb3_ref, E_ref,
          lsm_ref, aout_ref, pg_ref, rec_ref,
          r2b_ref, r3b_ref, sem2, sem3):
    c2 = pltpu.make_async_copy(R2_ref, r2b_ref, sem2)
    c2.start()
    c3 = pltpu.make_async_copy(R3_ref, r3b_ref, sem3)
    c3.start()
    imgP = imgP_ref[:]
    xposP = jax.lax.broadcasted_iota(jnp.int32, (1, BX), 1) % SIMG
    yposP = jax.lax.broadcasted_iota(jnp.int32, (SIMG, 1), 0)

    gxP = jnp.where(xposP < SIMG - 1, _pshift(imgP, 0, 1) - imgP, 0.0)
    gyP = jnp.where(yposP < SIMG - 1, _pshift(imgP, 1, 0) - imgP, 0.0)
    aP = jnp.sqrt(gxP * gxP + gyP * gyP + 1e-12)
    pmy, pmx = gyP, gxP

    wts = [Wt1_ref[:], Wt2_ref[:], Wt3_ref[:], Wt4_ref[:], Wt5_ref[:]]
    bs = [b1_ref[:], b2_ref[:], b3_ref[:], b4_ref[:], b5_ref[:]]
    pxs = [Px1_ref[:], Px2_ref[:], Px3_ref[:], Px4_ref[:], Px5_ref[:]]
    pas = [Pa1_ref[:], Pa2_ref[:], Pa3_ref[:], Pa4_ref[:], Pa5_ref[:]]

    X = None
    A = None
    xg = None
    for i in range(5):
        s = 1 << i
        Y = SIMG >> i
        N = Y * BX
        ci, co = CONV_DIMS[i]
        Wt = wts[i]
        h = jnp.broadcast_to(bs[i], (co, N))
        for dy in range(-2, 3):
            if dy <= -Y or dy >= Y:
                continue
            for dx in range(-2, 3):
                if dx * s <= -SIMG or dx * s >= SIMG:
                    continue
                k = (dy + 2) * 5 + (dx + 2)
                Wk = Wt[k * co:(k + 1) * co, :]
                if dy == 0 and dx == 0:
                    piece = imgP.reshape(1, N) if i == 0 else X
                    h = h + jnp.dot(Wk, piece, preferred_element_type=F32)
                    continue
                dpy = pmy - _pshift(pmy, dy, dx * s)
                dpx = pmx - _pshift(pmx, dy, dx * s)
                gP = jnp.exp(-(dpy * dpy + dpx * dpx))
                if dx != 0:
                    xv = xposP + dx * s
                    gP = jnp.where((xv >= 0) & (xv < SIMG), gP, 0.0)
                if i == 0:
                    piece = (gP * _pshift(imgP, dy, dx * s)).reshape(1, N)
                    h = h + jnp.dot(Wk, piece, preferred_element_type=F32)
                else:
                    mk = jnp.dot(Wk, _lshift(X, dy * BX + dx * s),
                                 preferred_element_type=F32)
                    h = h + gP.reshape(1, N) * mk
        h = jax.nn.relu(h)
        if i > 0:
            h = h * A
        xg = _pool_ch(h, Y, s)
        if i == 4:
            xg = xg * 0.25
        if i == 0:
            agP = _pool_plane(jax.nn.relu(aP), s, scale=False)
            ag = agP.reshape(1, N // 2)
        else:
            ag = _pool_ch(jax.nn.relu(A), Y, s)
        pmy = _pool_plane(pmy, s)
        pmx = _pool_plane(pmx, s)
        X = jax.nn.relu(jnp.dot(pxs[i], xg, preferred_element_type=F32))
        A = jax.nn.sigmoid(jnp.dot(pas[i], ag, preferred_element_type=F32))

    E = E_ref[:]
    logits = jnp.dot(X, E, preferred_element_type=F32)
    m = jnp.max(logits, axis=0, keepdims=True)
    z = logits - m
    lsm = z - jnp.log(jnp.sum(jnp.exp(z), axis=0, keepdims=True))
    aT = jnp.dot(A, E, preferred_element_type=F32)
    xgT = jnp.dot(xg, E, preferred_element_type=F32)
    pgy = jnp.dot(pmy.reshape(1, BX), E, preferred_element_type=F32)
    pgx = jnp.dot(pmx.reshape(1, BX), E, preferred_element_type=F32)

    rec_in = jnp.concatenate([xgT, aT] + [pgy, pgx] * 10, axis=0)
    rec_inT = rec_in.T
    r1 = jax.nn.relu(jnp.dot(rec_inT, R1_ref[:],
                             preferred_element_type=F32) + rb1_ref[:])
    c2.wait()
    r2 = jax.nn.relu(jnp.dot(r1, r2b_ref[:],
                             preferred_element_type=F32) + rb2_ref[:])
    c3.wait()
    r3 = jax.nn.sigmoid(jnp.dot(r2, r3b_ref[:],
                                preferred_element_type=F32) + rb3_ref[:])

    lsm_ref[:] = lsm
    aout_ref[:] = aT
    pg_ref[:] = jnp.concatenate([pgy, pgx], axis=0)
    rec_ref[:] = r3


def kernel(img_batch, W1, b1, W2, b2, W3, b3, W4, b4, W5, b5,
           Px1, Pa1, Px2, Pa2, Px3, Pa3, Px4, Pa4, Px5, Pa5,
           R1, rb1, R2, rb2, R3, rb3):
    B = img_batch.shape[0]
    imgP = img_batch[..., 0].transpose(1, 0, 2).reshape(SIMG, BX)

    flat_w = []
    for W, b in ((W1, b1), (W2, b2), (W3, b3), (W4, b4), (W5, b5)):
        k, ci, co = W.shape
        flat_w.append(W.transpose(0, 2, 1).reshape(k * co, ci))
        flat_w.append(b.reshape(co, 1))
    pool_w = [Px1.T * 0.25, Pa1.T * 0.25, Px2.T * 0.25, Pa2.T * 0.25,
              Px3.T * 0.25, Pa3.T * 0.25, Px4.T * 0.25, Pa4.T * 0.25,
              Px5.T, Pa5.T * 0.25]
    n = jnp.arange(BX)[:, None]
    E = (n == (jnp.arange(NIMG) * SIMG)[None, :]).astype(F32)

    vmem = pl.BlockSpec(memory_space=pltpu.MemorySpace.VMEM)
    hbm = pl.BlockSpec(memory_space=pltpu.MemorySpace.HBM)
    in_specs = [vmem] * 28
    in_specs[23] = hbm
    in_specs[25] = hbm
    lsm_t, a_t, pg_t, rec = pl.pallas_call(
        _body,
        in_specs=in_specs,
        scratch_shapes=[
            pltpu.VMEM((512, 1024), F32), pltpu.VMEM((1024, 784), F32),
            pltpu.SemaphoreType.DMA, pltpu.SemaphoreType.DMA,
        ],
        out_shape=(
            jax.ShapeDtypeStruct((10, NIMG), F32),
            jax.ShapeDtypeStruct((10, NIMG), F32),
            jax.ShapeDtypeStruct((2, NIMG), F32),
            jax.ShapeDtypeStruct((NIMG, 784), F32),
        ),
    )(imgP, *flat_w, *pool_w,
      R1, rb1.reshape(1, -1), R2, rb2.reshape(1, -1),
      R3, rb3.reshape(1, -1), E)

    pg = pg_t.T
    pose = jnp.broadcast_to(pg.reshape(B, 1, 1, 1, 2), (B, 1, 1, 10, 2))
    return (lsm_t.T, a_t.T, pose, rec)


# speedup vs baseline: 272.7186x; 1.1462x over previous
import jax
import jax.numpy as jnp
from jax.experimental import pallas as pl
from jax.experimental.pallas import tpu as pltpu

BX = 1024
NIMG = 32
SIMG = 32
F32 = jnp.float32

CONV_DIMS = [(1, 16), (16, 16), (32, 32), (32, 32), (64, 64)]


def _cdot(Wm, V):
    return jax.lax.dot_general(Wm, V, (((0,), (0,)), ((), ())),
                               preferred_element_type=F32)


def _pshift(P, dy, dl):
    Y, L = P.shape
    P2 = jnp.pad(P, ((max(-dy, 0), max(dy, 0)), (max(-dl, 0), max(dl, 0))))
    return P2[max(dy, 0):max(dy, 0) + Y, max(dl, 0):max(dl, 0) + L]


def _lshift(t, d):
    C, N = t.shape
    t2 = jnp.pad(t, ((0, 0), (max(-d, 0), max(d, 0))))
    return t2[:, max(d, 0):max(d, 0) + N]


def _pool_ch(t, Y, s):
    tx = t + _lshift(t, s)
    ty = tx + _lshift(tx, BX)
    rows = [ty[:, (2 * j) * BX:(2 * j) * BX + BX] for j in range(Y // 2)]
    return jnp.concatenate(rows, axis=1) * 0.25


def _pool_plane(P, s):
    Y = P.shape[0]
    Px_ = P + _pshift(P, 0, s)
    Py_ = Px_ + _pshift(Px_, 1, 0)
    rows = [Py_[2 * j:2 * j + 1] for j in range(Y // 2)]
    return jnp.concatenate(rows, axis=0) * 0.25


def _body(imgP_ref,
          W1_ref, b1_ref, W2_ref, b2_ref, W3_ref, b3_ref,
          W4_ref, b4_ref, W5_ref, b5_ref,
          Px1_ref, Pa1_ref, Px2_ref, Pa2_ref, Px3_ref, Pa3_ref,
          Px4_ref, Pa4_ref, Px5_ref, Pa5_ref,
          R1_ref, rb1_ref, R2_ref, rb2_ref, R3_ref, rb3_ref,
          lsm_ref, aout_ref, pg_ref, rec_ref,
          r2b_ref, r3b_ref, sem2, sem3):
    c2 = pltpu.make_async_copy(R2_ref, r2b_ref, sem2)
    c2.start()
    c3 = pltpu.make_async_copy(R3_ref, r3b_ref, sem3)
    c3.start()
    imgP = imgP_ref[:]
    xposP = jax.lax.broadcasted_iota(jnp.int32, (1, BX), 1) % SIMG
    yposP = jax.lax.broadcasted_iota(jnp.int32, (SIMG, 1), 0)

    gxP = jnp.where(xposP < SIMG - 1, _pshift(imgP, 0, 1) - imgP, 0.0)
    gyP = jnp.where(yposP < SIMG - 1, _pshift(imgP, 1, 0) - imgP, 0.0)
    aP = jnp.sqrt(gxP * gxP + gyP * gyP + 1e-12)
    pmy, pmx = gyP, gxP

    wts = [W1_ref[:], W2_ref[:], W3_ref[:], W4_ref[:], W5_ref[:]]
    bs = [b1_ref[:], b2_ref[:], b3_ref[:], b4_ref[:], b5_ref[:]]
    pxs = [Px1_ref[:], Px2_ref[:], Px3_ref[:], Px4_ref[:], Px5_ref[:]]
    pas = [Pa1_ref[:], Pa2_ref[:], Pa3_ref[:], Pa4_ref[:], Pa5_ref[:]]

    X = None
    A = None
    xg = None
    for i in range(5):
        s = 1 << i
        Y = SIMG >> i
        N = Y * BX
        ci, co = CONV_DIMS[i]
        W3d = wts[i]
        h = jnp.broadcast_to(bs[i], (co, N))
        for dy in range(-2, 3):
            if dy <= -Y or dy >= Y:
                continue
            for dx in range(-2, 3):
                if dx * s <= -SIMG or dx * s >= SIMG:
                    continue
                k = (dy + 2) * 5 + (dx + 2)
                Wk = W3d[k]
                if dy == 0 and dx == 0:
                    piece = imgP.reshape(1, N) if i == 0 else X
                    h = h + _cdot(Wk, piece)
                    continue
                dpy = pmy - _pshift(pmy, dy, dx * s)
                dpx = pmx - _pshift(pmx, dy, dx * s)
                gP = jnp.exp(-(dpy * dpy + dpx * dpx))
                if dx != 0:
                    xv = xposP + dx * s
                    gP = jnp.where((xv >= 0) & (xv < SIMG), gP, 0.0)
                if i == 0:
                    piece = (gP * _pshift(imgP, dy, dx * s)).reshape(1, N)
                    h = h + _cdot(Wk, piece)
                else:
                    mk = _cdot(Wk, _lshift(X, dy * BX + dx * s))
                    h = h + gP.reshape(1, N) * mk
        h = jax.nn.relu(h)
        if i > 0:
            h = h * A
        xg = _pool_ch(h, Y, s)
        if i == 0:
            agP = _pool_plane(jax.nn.relu(aP), s)
            ag = agP.reshape(1, N // 2)
        else:
            ag = _pool_ch(jax.nn.relu(A), Y, s)
        pmy = _pool_plane(pmy, s)
        pmx = _pool_plane(pmx, s)
        X = jax.nn.relu(_cdot(pxs[i], xg))
        A = jax.nn.sigmoid(_cdot(pas[i], ag))

    E = (jax.lax.broadcasted_iota(jnp.int32, (BX, NIMG), 0)
         == jax.lax.broadcasted_iota(jnp.int32, (BX, NIMG), 1) * SIMG
         ).astype(F32)
    logits = jnp.dot(X, E, preferred_element_type=F32)
    m = jnp.max(logits, axis=0, keepdims=True)
    z = logits - m
    lsm = z - jnp.log(jnp.sum(jnp.exp(z), axis=0, keepdims=True))
    aT = jnp.dot(A, E, preferred_element_type=F32)
    xgT = jnp.dot(xg, E, preferred_element_type=F32)
    pgy = jnp.dot(pmy.reshape(1, BX), E, preferred_element_type=F32)
    pgx = jnp.dot(pmx.reshape(1, BX), E, preferred_element_type=F32)

    rec_in = jnp.concatenate([xgT, aT] + [pgy, pgx] * 10, axis=0)
    rec_inT = rec_in.T
    r1 = jax.nn.relu(jnp.dot(rec_inT, R1_ref[:],
                             preferred_element_type=F32) + rb1_ref[:])
    c2.wait()
    r2 = jax.nn.relu(jnp.dot(r1, r2b_ref[:],
                             preferred_element_type=F32) + rb2_ref[:])
    c3.wait()
    r3 = jax.nn.sigmoid(jnp.dot(r2, r3b_ref[:],
                                preferred_element_type=F32) + rb3_ref[:])

    lsm_ref[:] = lsm.T
    aout_ref[:] = aT.T
    pg_ref[:] = jnp.concatenate([pgy, pgx], axis=0).T
    rec_ref[:] = r3


def kernel(img_batch, W1, b1, W2, b2, W3, b3, W4, b4, W5, b5,
           Px1, Pa1, Px2, Pa2, Px3, Pa3, Px4, Pa4, Px5, Pa5,
           R1, rb1, R2, rb2, R3, rb3):
    B = img_batch.shape[0]
    imgP = img_batch[..., 0].transpose(1, 0, 2).reshape(SIMG, BX)

    flat_w = []
    for W, b in ((W1, b1), (W2, b2), (W3, b3), (W4, b4), (W5, b5)):
        co = W.shape[2]
        flat_w.append(W)
        flat_w.append(b.reshape(co, 1))
    pool_w = [Px1, Pa1, Px2, Pa2, Px3, Pa3, Px4, Pa4, Px5, Pa5]

    vmem = pl.BlockSpec(memory_space=pltpu.MemorySpace.VMEM)
    hbm = pl.BlockSpec(memory_space=pltpu.MemorySpace.HBM)
    in_specs = [vmem] * 27
    in_specs[23] = hbm
    in_specs[25] = hbm
    lsm, a_out, pg, rec = pl.pallas_call(
        _body,
        in_specs=in_specs,
        scratch_shapes=[
            pltpu.VMEM((512, 1024), F32), pltpu.VMEM((1024, 784), F32),
            pltpu.SemaphoreType.DMA, pltpu.SemaphoreType.DMA,
        ],
        out_shape=(
            jax.ShapeDtypeStruct((NIMG, 10), F32),
            jax.ShapeDtypeStruct((NIMG, 10), F32),
            jax.ShapeDtypeStruct((NIMG, 2), F32),
            jax.ShapeDtypeStruct((NIMG, 784), F32),
        ),
    )(imgP, *flat_w, *pool_w,
      R1, rb1.reshape(1, -1), R2, rb2.reshape(1, -1),
      R3, rb3.reshape(1, -1))

    pose = jnp.broadcast_to(pg.reshape(B, 1, 1, 1, 2), (B, 1, 1, 10, 2))
    return (lsm, a_out, pose, rec)
